# speedup vs baseline: 1.4538x; 1.4538x over previous
import jax
import jax.numpy as jnp
from jax.experimental import pallas as pl
from jax.experimental.pallas import tpu as pltpu

_ROWS = 16384
_LANES = 128
_BR = 2048
_NB = _ROWS // _BR


def _loss_kernel(wmax_ref, wmin_ref, wptv_ref,
                 pred_ref, dose_ref, mask_ref, ptv_ref, oar_ref,
                 out_ref,
                 s_ref, mx_ref, mn_ref):
    b = pl.program_id(0)
    r = pl.program_id(1)

    @pl.when(jnp.logical_and(b == 0, r == 0))
    def _init():
        s_ref[...] = jnp.zeros_like(s_ref)
        mx_ref[...] = jnp.full_like(mx_ref, -jnp.inf)
        mn_ref[...] = jnp.full_like(mn_ref, jnp.inf)

    pred = pred_ref[0]
    dose = dose_ref[0, 0]
    mask = mask_ref[0, 0]
    ptv = ptv_ref[0]

    m = (mask > 0.0).astype(jnp.float32)
    mp = (ptv > 0.0).astype(jnp.float32)
    oar_sum = jnp.sum(oar_ref[0], axis=0)
    mo = (oar_sum > 0.0).astype(jnp.float32)

    d = jnp.abs(pred - dose)

    def tile_sum(x):
        return jnp.sum(x.reshape(_BR // 8, 8, _LANES), axis=0)

    s_ref[0] += tile_sum(d * m)
    s_ref[1] += tile_sum(m)
    s_ref[2] += tile_sum(d * mp)
    s_ref[3] += tile_sum(mp)
    s_ref[4] += tile_sum(d * mo)
    s_ref[5] += tile_sum(mo)

    neg_inf = jnp.float32(-jnp.inf)
    pos_inf = jnp.float32(jnp.inf)
    mb = mask > 0.0

    def tile_max(x):
        return jnp.max(x.reshape(_BR // 8, 8, _LANES), axis=0)

    def tile_min(x):
        return jnp.min(x.reshape(_BR // 8, 8, _LANES), axis=0)

    mx_ref[0] = jnp.maximum(mx_ref[0], tile_max(jnp.where(mb, dose, neg_inf)))
    mx_ref[1] = jnp.maximum(mx_ref[1], tile_max(jnp.where(mb, pred, neg_inf)))
    mn_ref[0] = jnp.minimum(mn_ref[0], tile_min(jnp.where(mb, dose, pos_inf)))
    mn_ref[1] = jnp.minimum(mn_ref[1], tile_min(jnp.where(mb, pred, pos_inf)))

    @pl.when(jnp.logical_and(b == pl.num_programs(0) - 1,
                             r == pl.num_programs(1) - 1))
    def _finalize():
        l1_num = jnp.sum(s_ref[0])
        l1_den = jnp.sum(s_ref[1])
        ptv_num = jnp.sum(s_ref[2])
        ptv_den = jnp.sum(s_ref[3])
        oar_num = jnp.sum(s_ref[4])
        oar_den = jnp.sum(s_ref[5])
        dose_max = jnp.max(mx_ref[0])
        pred_max = jnp.max(mx_ref[1])
        dose_min = jnp.min(mn_ref[0])
        pred_min = jnp.min(mn_ref[1])

        max_pen = jnp.maximum(pred_max - dose_max, 0.0) ** 2
        min_pen = jnp.maximum(dose_min - pred_min, 0.0) ** 2
        total = (l1_num / l1_den
                 + wptv_ref[0] * (ptv_num / ptv_den)
                 + oar_num / oar_den
                 + wmax_ref[0] * max_pen
                 + wmin_ref[0] * min_pen)
        out_ref[0] = total


@jax.jit
def kernel(pred, gt, PTVs, OAR, max_dose_weight, min_dose_weight, PTV_weight):
    pred3 = pred.reshape(2, _ROWS, _LANES)
    gt4 = gt.reshape(2, 2, _ROWS, _LANES)
    ptv3 = PTVs.reshape(2, _ROWS, _LANES)
    oar4 = OAR.reshape(2, 7, _ROWS, _LANES)

    grid = (2, _NB)

    out = pl.pallas_call(
        _loss_kernel,
        grid=grid,
        in_specs=[
            pl.BlockSpec(memory_space=pltpu.SMEM),
            pl.BlockSpec(memory_space=pltpu.SMEM),
            pl.BlockSpec(memory_space=pltpu.SMEM),
            pl.BlockSpec((1, _BR, _LANES), lambda b, r: (b, r, 0)),
            pl.BlockSpec((1, 1, _BR, _LANES), lambda b, r: (0, b, r, 0)),
            pl.BlockSpec((1, 1, _BR, _LANES), lambda b, r: (1, b, r, 0)),
            pl.BlockSpec((1, _BR, _LANES), lambda b, r: (b, r, 0)),
            pl.BlockSpec((1, 7, _BR, _LANES), lambda b, r: (b, 0, r, 0)),
        ],
        out_specs=pl.BlockSpec(memory_space=pltpu.SMEM),
        out_shape=jax.ShapeDtypeStruct((1,), jnp.float32),
        scratch_shapes=[
            pltpu.VMEM((6, 8, _LANES), jnp.float32),
            pltpu.VMEM((2, 8, _LANES), jnp.float32),
            pltpu.VMEM((2, 8, _LANES), jnp.float32),
        ],
    )(
        max_dose_weight.reshape(1), min_dose_weight.reshape(1),
        PTV_weight.reshape(1),
        pred3, gt4, gt4, ptv3, oar4,
    )
    return out[0]
